# baseline (speedup 1.0000x reference)
import jax
import jax.numpy as jnp
from jax import lax
from jax.experimental import pallas as pl
from jax.experimental.pallas import tpu as pltpu
from jax.experimental.pallas import tpu_sc as plsc

_H = 32
_W = 32
_B = 16
_D = 256
_RG = 4
_BG = 4


def _pos_emb_body(row_hbm, col_hbm, out_hbm, slab, bsem, osem):
    w = lax.axis_index("s") * 2 + lax.axis_index("c")
    ig = w // _BG
    bg = w % _BG
    build = [
        pltpu.async_copy(
            row_hbm.at[ig * _RG + r], slab.at[r, 0, pl.ds(0, _D)], bsem
        )
        for r in range(_RG)
    ] + [
        pltpu.async_copy(
            col_hbm.at[pl.ds(0, _W)], slab.at[r, :, pl.ds(_D, _D)], bsem
        )
        for r in range(_RG)
    ]
    for c in build:
        c.wait()
    for r in range(_RG):
        vregs = [slab[r, 0, pl.ds(k * 16, 16)] for k in range(_D // 16)]
        for j in range(1, _W):
            for k in range(_D // 16):
                slab[r, j, pl.ds(k * 16, 16)] = vregs[k]
    copies = [
        pltpu.async_copy(
            slab, out_hbm.at[bg * _BG + b, pl.ds(ig * _RG, _RG)], osem
        )
        for b in range(_BG)
    ]
    for c in copies:
        c.wait()


def kernel(tensor, row_embed, col_embed):
    del tensor
    mesh = plsc.VectorSubcoreMesh(core_axis_name="c", subcore_axis_name="s")
    f = pl.kernel(
        _pos_emb_body,
        out_type=jax.ShapeDtypeStruct((_B, _H, _W, 2 * _D), jnp.float32),
        mesh=mesh,
        scratch_types=[
            pltpu.VMEM((_RG, _W, 2 * _D), jnp.float32),
            pltpu.SemaphoreType.DMA,
            pltpu.SemaphoreType.DMA,
        ],
    )
    return f(row_embed, col_embed)
